# baseline (speedup 1.0000x reference)
import functools

import jax
import jax.numpy as jnp
from jax import lax
from jax.experimental import pallas as pl
from jax.experimental.pallas import tpu as pltpu
from jax.experimental.pallas import tpu_sc as plsc

Q = 2048
H = 16
BAND = 240
TAB_ROWS = 512
LAG = 8

_THRESHOLDS = (19, 21, 24, 27, 31, 35, 40, 46, 52, 59, 67, 77, 87, 99, 113)


def _table_kernel(emb_smem, tab_ref):
    h = pl.program_id(0)
    r = lax.broadcasted_iota(jnp.int32, (256, 128), 0)
    c = lax.broadcasted_iota(jnp.int32, (256, 128), 1)
    d = jnp.maximum(r - c, 0)
    large = jnp.full_like(d, 16)
    for thr in _THRESHOLDS:
        large = large + (d >= thr).astype(jnp.int32)
    b = jnp.where(d < 16, d, large)
    acc = jnp.zeros((256, 128), jnp.float32)
    for bb in range(32):
        acc = jnp.where(b == bb, emb_smem[bb, h], acc)
    tab_ref[0, :256, :] = acc
    tab_ref[0, 256:384, :] = jnp.full((128, 128), emb_smem[31, h], jnp.float32)
    tab_ref[0, 384:, :] = jnp.full((128, 128), emb_smem[0, h], jnp.float32)


def _sc_fill_kernel(tab_hbm, out_hbm, tab_v, sem):
    c_ax = lax.axis_index("c")
    s_ax = lax.axis_index("s")
    wid = s_ax * 2 + c_ax
    h = wid // 2
    half = wid - 2 * h
    pltpu.sync_copy(tab_hbm.at[h], tab_v)

    def emit(c_tiles):
        descs = []

        def fire(src_row, n_rows, dst_row, col):
            descs.append(pltpu.async_copy(
                tab_v.at[pl.ds(src_row, n_rows), :],
                out_hbm.at[0, h, pl.ds(dst_row, n_rows), pl.ds(col, 128)],
                sem,
            ))
            if len(descs) > LAG:
                descs[len(descs) - LAG - 1].wait()

        for C in c_tiles:
            col = 128 * C
            bh = min(BAND, Q - col)
            fire(0, bh, col, col)
            for r0 in range(0, col, 128):
                fire(384, 128, r0, col)
            r0 = col + BAND
            while r0 < Q:
                n = min(128, Q - r0)
                fire(256, n, r0, col)
                r0 += n
        for dsc in descs[max(0, len(descs) - LAG):]:
            dsc.wait()

    @pl.when(half == 0)
    def _():
        emit(range(8))

    @pl.when(half == 1)
    def _():
        emit(range(8, 16))


def kernel(query_length, key_length, embeddings):
    del query_length, key_length

    tab = pl.pallas_call(
        _table_kernel,
        grid=(H,),
        in_specs=[pl.BlockSpec(memory_space=pltpu.SMEM)],
        out_specs=pl.BlockSpec((1, TAB_ROWS, 128), lambda h: (h, 0, 0)),
        out_shape=jax.ShapeDtypeStruct((H, TAB_ROWS, 128), jnp.float32),
    )(embeddings)

    sc_fill = functools.partial(
        pl.kernel,
        mesh=plsc.VectorSubcoreMesh(core_axis_name="c", subcore_axis_name="s",
                                    num_cores=2),
        out_type=jax.ShapeDtypeStruct((1, H, Q, Q), jnp.float32),
        scratch_types=[
            pltpu.VMEM((TAB_ROWS, 128), jnp.float32),
            pltpu.SemaphoreType.DMA,
        ],
        compiler_params=pltpu.CompilerParams(use_tc_tiling_on_sc=True),
    )(_sc_fill_kernel)

    return sc_fill(tab)


# speedup vs baseline: 1.0123x; 1.0123x over previous
import functools

import jax
import jax.numpy as jnp
from jax import lax
from jax.experimental import pallas as pl
from jax.experimental.pallas import tpu as pltpu
from jax.experimental.pallas import tpu_sc as plsc

Q = 2048
H = 16
BAND = 240
TAB_ROWS = 512
LAG = 8

_THRESHOLDS = (19, 21, 24, 27, 31, 35, 40, 46, 52, 59, 67, 77, 87, 99, 113)


def _table_kernel(emb_smem, tab_ref):
    h = pl.program_id(0)
    u = lax.broadcasted_iota(jnp.int32, (8, 512), 1)
    d = jnp.maximum(384 - u, 0)
    large = jnp.full_like(d, 16)
    for thr in _THRESHOLDS:
        large = large + (d >= thr).astype(jnp.int32)
    b = jnp.where(d < 16, d, large)
    erev = jnp.zeros((8, 512), jnp.float32)
    for bb in range(32):
        erev = jnp.where(b == bb, emb_smem[bb, h], erev)
    for g in range(32):
        rolled = pltpu.roll(erev, (8 * g - 384) % 512, axis=1,
                            stride=1, stride_axis=0)
        tab_ref[0, 8 * g:8 * g + 8, :] = rolled[:, :128]
    tab_ref[0, 256:384, :] = jnp.full((128, 128), emb_smem[31, h], jnp.float32)
    tab_ref[0, 384:, :] = jnp.full((128, 128), emb_smem[0, h], jnp.float32)


def _sc_fill_kernel(tab_hbm, out_hbm, tab_v, sem):
    c_ax = lax.axis_index("c")
    s_ax = lax.axis_index("s")
    wid = s_ax * 2 + c_ax
    h = wid // 2
    half = wid - 2 * h
    pltpu.sync_copy(tab_hbm.at[h], tab_v)

    def emit(c_tiles):
        descs = []

        def fire(src_row, n_rows, dst_row, col):
            descs.append(pltpu.async_copy(
                tab_v.at[pl.ds(src_row, n_rows), :],
                out_hbm.at[0, h, pl.ds(dst_row, n_rows), pl.ds(col, 128)],
                sem,
            ))
            if len(descs) > LAG:
                descs[len(descs) - LAG - 1].wait()

        for C in c_tiles:
            col = 128 * C
            bh = min(BAND, Q - col)
            fire(0, bh, col, col)
            for r0 in range(0, col, 128):
                fire(384, 128, r0, col)
            r0 = col + BAND
            while r0 < Q:
                n = min(128, Q - r0)
                fire(256, n, r0, col)
                r0 += n
        for dsc in descs[max(0, len(descs) - LAG):]:
            dsc.wait()

    @pl.when(half == 0)
    def _():
        emit(range(8))

    @pl.when(half == 1)
    def _():
        emit(range(8, 16))


def kernel(query_length, key_length, embeddings):
    del query_length, key_length

    tab = pl.pallas_call(
        _table_kernel,
        grid=(H,),
        in_specs=[pl.BlockSpec(memory_space=pltpu.SMEM)],
        out_specs=pl.BlockSpec((1, TAB_ROWS, 128), lambda h: (h, 0, 0)),
        out_shape=jax.ShapeDtypeStruct((H, TAB_ROWS, 128), jnp.float32),
    )(embeddings)

    sc_fill = functools.partial(
        pl.kernel,
        mesh=plsc.VectorSubcoreMesh(core_axis_name="c", subcore_axis_name="s",
                                    num_cores=2),
        out_type=jax.ShapeDtypeStruct((1, H, Q, Q), jnp.float32),
        scratch_types=[
            pltpu.VMEM((TAB_ROWS, 128), jnp.float32),
            pltpu.SemaphoreType.DMA,
        ],
        compiler_params=pltpu.CompilerParams(use_tc_tiling_on_sc=True),
    )(_sc_fill_kernel)

    return sc_fill(tab)


# speedup vs baseline: 1.0138x; 1.0015x over previous
import functools

import jax
import jax.numpy as jnp
from jax import lax
from jax.experimental import pallas as pl
from jax.experimental.pallas import tpu as pltpu
from jax.experimental.pallas import tpu_sc as plsc

Q = 2048
H = 16
BAND = 240
TAB_ROWS = 512
LAG = 16

_THRESHOLDS = (19, 21, 24, 27, 31, 35, 40, 46, 52, 59, 67, 77, 87, 99, 113)


def _table_kernel(emb_smem, tab_ref):
    h = pl.program_id(0)
    u = lax.broadcasted_iota(jnp.int32, (8, 512), 1)
    d = jnp.maximum(384 - u, 0)
    large = jnp.full_like(d, 16)
    for thr in _THRESHOLDS:
        large = large + (d >= thr).astype(jnp.int32)
    b = jnp.where(d < 16, d, large)
    erev = jnp.zeros((8, 512), jnp.float32)
    for bb in range(32):
        erev = jnp.where(b == bb, emb_smem[bb, h], erev)
    for g in range(32):
        rolled = pltpu.roll(erev, (8 * g - 384) % 512, axis=1,
                            stride=1, stride_axis=0)
        tab_ref[0, 8 * g:8 * g + 8, :] = rolled[:, :128]
    tab_ref[0, 256:384, :] = jnp.full((128, 128), emb_smem[31, h], jnp.float32)
    tab_ref[0, 384:, :] = jnp.full((128, 128), emb_smem[0, h], jnp.float32)


def _sc_fill_kernel(tab_hbm, out_hbm, tab_v, sem):
    c_ax = lax.axis_index("c")
    s_ax = lax.axis_index("s")
    wid = s_ax * 2 + c_ax
    h = wid // 2
    half = wid - 2 * h
    pltpu.sync_copy(tab_hbm.at[h], tab_v)

    def emit(c_tiles):
        descs = []

        def fire(src_row, n_rows, dst_row, col):
            descs.append(pltpu.async_copy(
                tab_v.at[pl.ds(src_row, n_rows), :],
                out_hbm.at[0, h, pl.ds(dst_row, n_rows), pl.ds(col, 128)],
                sem,
            ))
            if len(descs) > LAG:
                descs[len(descs) - LAG - 1].wait()

        for C in c_tiles:
            col = 128 * C
            bh = min(BAND, Q - col)
            fire(0, bh, col, col)
            for r0 in range(0, col, 128):
                fire(384, 128, r0, col)
            r0 = col + BAND
            while r0 < Q:
                n = min(128, Q - r0)
                fire(256, n, r0, col)
                r0 += n
        for dsc in descs[max(0, len(descs) - LAG):]:
            dsc.wait()

    @pl.when(half == 0)
    def _():
        emit(range(8))

    @pl.when(half == 1)
    def _():
        emit(range(8, 16))


def kernel(query_length, key_length, embeddings):
    del query_length, key_length

    tab = pl.pallas_call(
        _table_kernel,
        grid=(H,),
        in_specs=[pl.BlockSpec(memory_space=pltpu.SMEM)],
        out_specs=pl.BlockSpec((1, TAB_ROWS, 128), lambda h: (h, 0, 0)),
        out_shape=jax.ShapeDtypeStruct((H, TAB_ROWS, 128), jnp.float32),
    )(embeddings)

    sc_fill = functools.partial(
        pl.kernel,
        mesh=plsc.VectorSubcoreMesh(core_axis_name="c", subcore_axis_name="s",
                                    num_cores=2),
        out_type=jax.ShapeDtypeStruct((1, H, Q, Q), jnp.float32),
        scratch_types=[
            pltpu.VMEM((TAB_ROWS, 128), jnp.float32),
            pltpu.SemaphoreType.DMA,
        ],
        compiler_params=pltpu.CompilerParams(use_tc_tiling_on_sc=True),
    )(_sc_fill_kernel)

    return sc_fill(tab)
